# Initial kernel scaffold; baseline (speedup 1.0000x reference)
import functools

import jax
import jax.numpy as jnp
from jax import lax
from jax.experimental import pallas as pl
from jax.experimental.pallas import tpu as pltpu
from jax.experimental.pallas import tpu_sc as plsc

N_NODES = 10000
N_EDGES = 160000
NCORE = 2
NSUB = 16
CHUNK = 80
EDGES_PER_SUB = N_EDGES // NSUB
NCHUNK = EDGES_PER_SUB // CHUNK
ROWS_PER_SUB = N_NODES // NSUB
DLO = 128
DY = 32

_MESH = plsc.VectorSubcoreMesh(core_axis_name="c", subcore_axis_name="s",
                               num_cores=NCORE, num_subcores=NSUB)


def _agg1_body(xlo, xhi, src3, dst3, zer128, zer16, ones16,
               zlo, zhi, recip16,
               acc, deg, srcv, dstv, buf, onesv, degbuf, recipbuf, sem):
    c = lax.axis_index("c")
    s = lax.axis_index("s")
    row0 = s * ROWS_PER_SUB

    pltpu.sync_copy(src3.at[s], srcv)
    pltpu.sync_copy(dst3.at[s], dstv)
    pltpu.sync_copy(ones16, onesv)

    pltpu.sync_copy(zer128, acc.at[pl.ds(row0, ROWS_PER_SUB)])
    pltpu.sync_copy(zer16, deg.at[pl.ds(row0, ROWS_PER_SUB)])
    plsc.subcore_barrier()

    def chunk_body(table):
        def body(j, carry):
            pltpu.async_copy(table.at[srcv.at[j]], buf, sem).wait()
            pltpu.sync_copy(buf, acc.at[dstv.at[j]], add=True)
            pltpu.sync_copy(onesv, deg.at[dstv.at[j]], add=True)
            return carry
        return body

    @pl.when(c == 0)
    def _():
        lax.fori_loop(0, NCHUNK, chunk_body(xlo), 0)

    @pl.when(c == 1)
    def _():
        lax.fori_loop(0, NCHUNK, chunk_body(xhi), 0)

    plsc.subcore_barrier()

    pltpu.sync_copy(deg.at[pl.ds(row0, ROWS_PER_SUB)], degbuf)

    def rbody(r, carry):
        d = degbuf[r, :]
        recipbuf[r, :] = 1.0 / jnp.maximum(d, 1.0)
        return carry
    lax.fori_loop(0, ROWS_PER_SUB, rbody, 0)

    @pl.when(c == 0)
    def _():
        pltpu.sync_copy(recipbuf, recip16.at[pl.ds(row0, ROWS_PER_SUB)])
        pltpu.sync_copy(acc.at[pl.ds(row0, ROWS_PER_SUB)],
                        zlo.at[pl.ds(row0, ROWS_PER_SUB)])

    @pl.when(c == 1)
    def _():
        pltpu.sync_copy(acc.at[pl.ds(row0, ROWS_PER_SUB)],
                        zhi.at[pl.ds(row0, ROWS_PER_SUB)])


_agg1 = pl.kernel(
    _agg1_body,
    out_type=(
        jax.ShapeDtypeStruct((N_NODES, DLO), jnp.float32),
        jax.ShapeDtypeStruct((N_NODES, DLO), jnp.float32),
        jax.ShapeDtypeStruct((N_NODES, 16), jnp.float32),
    ),
    mesh=_MESH,
    scratch_types=[
        pltpu.VMEM_SHARED((N_NODES, DLO), jnp.float32),
        pltpu.VMEM_SHARED((N_NODES, 16), jnp.float32),
        pltpu.VMEM((NCHUNK, CHUNK), jnp.int32),
        pltpu.VMEM((NCHUNK, CHUNK), jnp.int32),
        pltpu.VMEM((CHUNK, DLO), jnp.float32),
        pltpu.VMEM((CHUNK, 16), jnp.float32),
        pltpu.VMEM((ROWS_PER_SUB, 16), jnp.float32),
        pltpu.VMEM((ROWS_PER_SUB, 16), jnp.float32),
        pltpu.SemaphoreType.DMA,
    ],
)


def _agg2_body(ylo, yhi, src3, dst3, recip16, b2, zer32,
               olo, ohi,
               acc, srcv, dstv, buf, recipv, accbuf, outbuf, b2v, sem):
    c = lax.axis_index("c")
    s = lax.axis_index("s")
    row0 = s * ROWS_PER_SUB

    pltpu.sync_copy(src3.at[s], srcv)
    pltpu.sync_copy(dst3.at[s], dstv)
    pltpu.sync_copy(b2, b2v)
    pltpu.sync_copy(zer32, acc.at[pl.ds(row0, ROWS_PER_SUB)])
    plsc.subcore_barrier()

    def chunk_body(table):
        def body(j, carry):
            pltpu.async_copy(table.at[srcv.at[j]], buf, sem).wait()
            pltpu.sync_copy(buf, acc.at[dstv.at[j]], add=True)
            return carry
        return body

    @pl.when(c == 0)
    def _():
        lax.fori_loop(0, NCHUNK, chunk_body(ylo), 0)

    @pl.when(c == 1)
    def _():
        lax.fori_loop(0, NCHUNK, chunk_body(yhi), 0)

    plsc.subcore_barrier()

    pltpu.sync_copy(acc.at[pl.ds(row0, ROWS_PER_SUB)], accbuf)
    pltpu.sync_copy(recip16.at[pl.ds(row0, ROWS_PER_SUB)], recipv)
    b2a = b2v[pl.ds(c * DY, 16)]
    b2b = b2v[pl.ds(c * DY + 16, 16)]

    def rbody(r, carry):
        rv = recipv[r, :]
        outbuf[r, pl.ds(0, 16)] = accbuf[r, pl.ds(0, 16)] * rv + b2a
        outbuf[r, pl.ds(16, 16)] = accbuf[r, pl.ds(16, 16)] * rv + b2b
        return carry
    lax.fori_loop(0, ROWS_PER_SUB, rbody, 0)

    @pl.when(c == 0)
    def _():
        pltpu.sync_copy(outbuf, olo.at[pl.ds(row0, ROWS_PER_SUB)])

    @pl.when(c == 1)
    def _():
        pltpu.sync_copy(outbuf, ohi.at[pl.ds(row0, ROWS_PER_SUB)])


_agg2 = pl.kernel(
    _agg2_body,
    out_type=(
        jax.ShapeDtypeStruct((N_NODES, DY), jnp.float32),
        jax.ShapeDtypeStruct((N_NODES, DY), jnp.float32),
    ),
    mesh=_MESH,
    scratch_types=[
        pltpu.VMEM_SHARED((N_NODES, DY), jnp.float32),
        pltpu.VMEM((NCHUNK, CHUNK), jnp.int32),
        pltpu.VMEM((NCHUNK, CHUNK), jnp.int32),
        pltpu.VMEM((CHUNK, DY), jnp.float32),
        pltpu.VMEM((ROWS_PER_SUB, 16), jnp.float32),
        pltpu.VMEM((ROWS_PER_SUB, DY), jnp.float32),
        pltpu.VMEM((ROWS_PER_SUB, DY), jnp.float32),
        pltpu.VMEM((2 * DY,), jnp.float32),
        pltpu.SemaphoreType.DMA,
    ],
)


def _mlp_body(zlo_ref, zhi_ref, r16_ref, w1_ref, b1_ref, w2_ref,
              ylo_ref, yhi_ref):
    z = jnp.concatenate([zlo_ref[...], zhi_ref[...]], axis=1)
    r = r16_ref[...][:, 0:1]
    h = jnp.dot(z * r, w1_ref[...], preferred_element_type=jnp.float32)
    h = jnp.maximum(h + b1_ref[...], 0.0)
    y = jnp.dot(h, w2_ref[...], preferred_element_type=jnp.float32)
    ylo_ref[...] = y[:, :DY]
    yhi_ref[...] = y[:, DY:]


def _mlp(zlo, zhi, recip16, W1, b1, W2):
    blk = 2000
    grid = (N_NODES // blk,)
    return pl.pallas_call(
        _mlp_body,
        grid=grid,
        in_specs=[
            pl.BlockSpec((blk, DLO), lambda i: (i, 0)),
            pl.BlockSpec((blk, DLO), lambda i: (i, 0)),
            pl.BlockSpec((blk, 16), lambda i: (i, 0)),
            pl.BlockSpec((256, 256), lambda i: (0, 0)),
            pl.BlockSpec((1, 256), lambda i: (0, 0)),
            pl.BlockSpec((256, 2 * DY), lambda i: (0, 0)),
        ],
        out_specs=[
            pl.BlockSpec((blk, DY), lambda i: (i, 0)),
            pl.BlockSpec((blk, DY), lambda i: (i, 0)),
        ],
        out_shape=[
            jax.ShapeDtypeStruct((N_NODES, DY), jnp.float32),
            jax.ShapeDtypeStruct((N_NODES, DY), jnp.float32),
        ],
    )(zlo, zhi, recip16, W1, b1, W2)


@jax.jit
def kernel(inputs, edge_index, W1, b1, W2, b2):
    src = edge_index[0].astype(jnp.int32)
    dst = edge_index[1].astype(jnp.int32)
    src3 = src.reshape(NSUB, NCHUNK, CHUNK)
    dst3 = dst.reshape(NSUB, NCHUNK, CHUNK)

    xlo = inputs[:, :DLO]
    xhi = inputs[:, DLO:]
    zer128 = jnp.zeros((ROWS_PER_SUB, DLO), jnp.float32)
    zer16 = jnp.zeros((ROWS_PER_SUB, 16), jnp.float32)
    zer32 = jnp.zeros((ROWS_PER_SUB, DY), jnp.float32)
    ones16 = jnp.ones((CHUNK, 16), jnp.float32)

    zlo, zhi, recip16 = _agg1(xlo, xhi, src3, dst3, zer128, zer16, ones16)
    ylo, yhi = _mlp(zlo, zhi, recip16, W1, b1.reshape(1, -1), W2)
    olo, ohi = _agg2(ylo, yhi, src3, dst3, recip16, b2, zer32)
    return jnp.concatenate([olo, ohi], axis=1)


# speedup vs baseline: 4.7039x; 4.7039x over previous
import jax
import jax.numpy as jnp
from jax import lax
from jax.experimental import pallas as pl
from jax.experimental.pallas import tpu as pltpu
from jax.experimental.pallas import tpu_sc as plsc

N_NODES = 10000
N_PAD = 10240
N_EDGES = 160000
NCORE = 2
NSUB = 16
CHUNK = 80
EDGES_PER_SUB = N_EDGES // NSUB
NCHUNK = EDGES_PER_SUB // CHUNK
NBLK = 5
BCH = NCHUNK // NBLK
ROWS_PER_SUB = N_PAD // NSUB
DSLAB = 64
DY = 32

_MESH = plsc.VectorSubcoreMesh(core_axis_name="c", subcore_axis_name="s",
                               num_cores=NCORE, num_subcores=NSUB)


def _agg1_body(x0, x1, x2, x3, src4, dst4, zer64, zer16, ones16,
               z0, z1, z2, z3, recip16,
               acc, deg, srcv, dstv, buf, onesv, degbuf, sem):
    c = lax.axis_index("c")
    s = lax.axis_index("s")
    row0 = s * ROWS_PER_SUB
    stripe = pl.ds(row0, ROWS_PER_SUB)

    pltpu.sync_copy(ones16, onesv)

    def edge_loop(table, with_deg):
        def bbody(b, carry):
            pltpu.sync_copy(src4.at[s, b], srcv)
            pltpu.sync_copy(dst4.at[s, b], dstv)

            def body(j, carry2):
                pltpu.async_copy(table.at[srcv.at[j]], buf, sem).wait()
                pltpu.sync_copy(buf, acc.at[dstv.at[j]], add=True)
                if with_deg:
                    pltpu.sync_copy(onesv, deg.at[dstv.at[j]], add=True)
                return carry2
            return lax.fori_loop(0, BCH, body, carry)
        lax.fori_loop(0, NBLK, bbody, 0)

    for p in range(2):
        with_deg = p == 0
        pltpu.sync_copy(zer64, acc.at[stripe])
        if with_deg:
            pltpu.sync_copy(zer16, deg.at[stripe])
        plsc.subcore_barrier()

        @pl.when(c == 0)
        def _():
            edge_loop((x0, x1)[p], with_deg)

        @pl.when(c == 1)
        def _():
            edge_loop((x2, x3)[p], with_deg)

        plsc.subcore_barrier()

        zout = ((z0, z1), (z2, z3))

        @pl.when(c == 0)
        def _():
            pltpu.sync_copy(acc.at[stripe], zout[0][p].at[stripe])

        @pl.when(c == 1)
        def _():
            pltpu.sync_copy(acc.at[stripe], zout[1][p].at[stripe])

        plsc.subcore_barrier()

    @pl.when(c == 0)
    def _():
        pltpu.sync_copy(deg.at[stripe], degbuf)

        def rbody(r, carry):
            d = degbuf[r, :]
            degbuf[r, :] = 1.0 / jnp.maximum(d, 1.0)
            return carry
        lax.fori_loop(0, ROWS_PER_SUB, rbody, 0)
        pltpu.sync_copy(degbuf, recip16.at[stripe])


_agg1 = pl.kernel(
    _agg1_body,
    out_type=(
        jax.ShapeDtypeStruct((N_PAD, DSLAB), jnp.float32),
        jax.ShapeDtypeStruct((N_PAD, DSLAB), jnp.float32),
        jax.ShapeDtypeStruct((N_PAD, DSLAB), jnp.float32),
        jax.ShapeDtypeStruct((N_PAD, DSLAB), jnp.float32),
        jax.ShapeDtypeStruct((N_PAD, 16), jnp.float32),
    ),
    mesh=_MESH,
    compiler_params=pltpu.CompilerParams(use_tc_tiling_on_sc=False),
    scratch_types=[
        pltpu.VMEM_SHARED((N_PAD, DSLAB), jnp.float32),
        pltpu.VMEM_SHARED((N_PAD, 16), jnp.float32),
        pltpu.VMEM((BCH, CHUNK), jnp.int32),
        pltpu.VMEM((BCH, CHUNK), jnp.int32),
        pltpu.VMEM((CHUNK, DSLAB), jnp.float32),
        pltpu.VMEM((CHUNK, 16), jnp.float32),
        pltpu.VMEM((ROWS_PER_SUB, 16), jnp.float32),
        pltpu.SemaphoreType.DMA,
    ],
)


def _agg2_body(ylo, yhi, src4, dst4, recip16, b2, zer32,
               olo, ohi,
               acc, srcv, dstv, buf, recipv, accbuf, b2v, sem):
    c = lax.axis_index("c")
    s = lax.axis_index("s")
    row0 = s * ROWS_PER_SUB
    stripe = pl.ds(row0, ROWS_PER_SUB)

    pltpu.sync_copy(b2, b2v)
    pltpu.sync_copy(zer32, acc.at[stripe])
    plsc.subcore_barrier()

    def edge_loop(table):
        def bbody(b, carry):
            pltpu.sync_copy(src4.at[s, b], srcv)
            pltpu.sync_copy(dst4.at[s, b], dstv)

            def body(j, carry2):
                pltpu.async_copy(table.at[srcv.at[j]], buf, sem).wait()
                pltpu.sync_copy(buf, acc.at[dstv.at[j]], add=True)
                return carry2
            return lax.fori_loop(0, BCH, body, carry)
        lax.fori_loop(0, NBLK, bbody, 0)

    @pl.when(c == 0)
    def _():
        edge_loop(ylo)

    @pl.when(c == 1)
    def _():
        edge_loop(yhi)

    plsc.subcore_barrier()

    pltpu.sync_copy(acc.at[stripe], accbuf)
    pltpu.sync_copy(recip16.at[stripe], recipv)
    b2a = b2v[pl.ds(c * DY, 16)]
    b2b = b2v[pl.ds(c * DY + 16, 16)]

    def rbody(r, carry):
        rv = recipv[r, :]
        accbuf[r, pl.ds(0, 16)] = accbuf[r, pl.ds(0, 16)] * rv + b2a
        accbuf[r, pl.ds(16, 16)] = accbuf[r, pl.ds(16, 16)] * rv + b2b
        return carry
    lax.fori_loop(0, ROWS_PER_SUB, rbody, 0)

    @pl.when(c == 0)
    def _():
        pltpu.sync_copy(accbuf, olo.at[stripe])

    @pl.when(c == 1)
    def _():
        pltpu.sync_copy(accbuf, ohi.at[stripe])


_agg2 = pl.kernel(
    _agg2_body,
    out_type=(
        jax.ShapeDtypeStruct((N_PAD, DY), jnp.float32),
        jax.ShapeDtypeStruct((N_PAD, DY), jnp.float32),
    ),
    mesh=_MESH,
    compiler_params=pltpu.CompilerParams(use_tc_tiling_on_sc=False),
    scratch_types=[
        pltpu.VMEM_SHARED((N_PAD, DY), jnp.float32),
        pltpu.VMEM((BCH, CHUNK), jnp.int32),
        pltpu.VMEM((BCH, CHUNK), jnp.int32),
        pltpu.VMEM((CHUNK, DY), jnp.float32),
        pltpu.VMEM((ROWS_PER_SUB, 16), jnp.float32),
        pltpu.VMEM((ROWS_PER_SUB, DY), jnp.float32),
        pltpu.VMEM((2 * DY,), jnp.float32),
        pltpu.SemaphoreType.DMA,
    ],
)


def _mlp_body(z0_ref, z1_ref, z2_ref, z3_ref, r16_ref, w1_ref, b1_ref,
              w2_ref, ylo_ref, yhi_ref):
    z = jnp.concatenate(
        [z0_ref[...], z1_ref[...], z2_ref[...], z3_ref[...]], axis=1)
    r = r16_ref[...][:, 0:1]
    h = jnp.dot(z * r, w1_ref[...], preferred_element_type=jnp.float32)
    h = jnp.maximum(h + b1_ref[...], 0.0)
    y = jnp.dot(h, w2_ref[...], preferred_element_type=jnp.float32)
    ylo_ref[...] = y[:, :DY]
    yhi_ref[...] = y[:, DY:]


def _mlp(z0, z1, z2, z3, recip16, W1, b1, W2):
    blk = 2048
    grid = (N_PAD // blk,)
    return pl.pallas_call(
        _mlp_body,
        grid=grid,
        in_specs=[
            pl.BlockSpec((blk, DSLAB), lambda i: (i, 0)),
            pl.BlockSpec((blk, DSLAB), lambda i: (i, 0)),
            pl.BlockSpec((blk, DSLAB), lambda i: (i, 0)),
            pl.BlockSpec((blk, DSLAB), lambda i: (i, 0)),
            pl.BlockSpec((blk, 16), lambda i: (i, 0)),
            pl.BlockSpec((256, 256), lambda i: (0, 0)),
            pl.BlockSpec((1, 256), lambda i: (0, 0)),
            pl.BlockSpec((256, 2 * DY), lambda i: (0, 0)),
        ],
        out_specs=[
            pl.BlockSpec((blk, DY), lambda i: (i, 0)),
            pl.BlockSpec((blk, DY), lambda i: (i, 0)),
        ],
        out_shape=[
            jax.ShapeDtypeStruct((N_PAD, DY), jnp.float32),
            jax.ShapeDtypeStruct((N_PAD, DY), jnp.float32),
        ],
    )(z0, z1, z2, z3, recip16, W1, b1, W2)


@jax.jit
def kernel(inputs, edge_index, W1, b1, W2, b2):
    src = edge_index[0].astype(jnp.int32)
    dst = edge_index[1].astype(jnp.int32)
    src4 = src.reshape(NSUB, NBLK, BCH, CHUNK)
    dst4 = dst.reshape(NSUB, NBLK, BCH, CHUNK)

    xs = [inputs[:, i * DSLAB:(i + 1) * DSLAB] for i in range(4)]
    zer64 = jnp.zeros((ROWS_PER_SUB, DSLAB), jnp.float32)
    zer16 = jnp.zeros((ROWS_PER_SUB, 16), jnp.float32)
    zer32 = jnp.zeros((ROWS_PER_SUB, DY), jnp.float32)
    ones16 = jnp.ones((CHUNK, 16), jnp.float32)

    z0, z1, z2, z3, recip16 = _agg1(
        xs[0], xs[1], xs[2], xs[3], src4, dst4, zer64, zer16, ones16)
    ylo, yhi = _mlp(z0, z1, z2, z3, recip16, W1, b1.reshape(1, -1), W2)
    olo, ohi = _agg2(ylo, yhi, src4, dst4, recip16, b2, zer32)
    return jnp.concatenate([olo, ohi], axis=1)[:N_NODES]


# speedup vs baseline: 5.8299x; 1.2394x over previous
import jax
import jax.numpy as jnp
from jax import lax
from jax.experimental import pallas as pl
from jax.experimental.pallas import tpu as pltpu
from jax.experimental.pallas import tpu_sc as plsc

N_NODES = 10000
N_PAD = 10240
N_EDGES = 160000
NCORE = 2
NSUB = 16
CHUNK = 80
EDGES_PER_SUB = N_EDGES // NSUB
NCHUNK = EDGES_PER_SUB // CHUNK
NPAIR = (NCHUNK - 1) // 2
ROWS_PER_SUB = N_PAD // NSUB
DSLAB = 64
DY = 32

_MESH = plsc.VectorSubcoreMesh(core_axis_name="c", subcore_axis_name="s",
                               num_cores=NCORE, num_subcores=NSUB)


def _agg1_body(x0, x1, x2, x3, src3, dst3, zer64, zer16, ones16,
               z0, z1, z2, z3, recip16,
               acc, deg, srcv, dstv, buf0, buf1, onesv, degbuf, sem0, sem1):
    c = lax.axis_index("c")
    s = lax.axis_index("s")
    row0 = s * ROWS_PER_SUB
    stripe = pl.ds(row0, ROWS_PER_SUB)

    pltpu.sync_copy(ones16, onesv)
    pltpu.sync_copy(src3.at[s], srcv)
    pltpu.sync_copy(dst3.at[s], dstv)

    def edge_loop(table, with_deg):
        def fire(j, buf, sem):
            pltpu.async_copy(table.at[srcv.at[j]], buf, sem)

        def wait(j, buf, sem):
            pltpu.make_async_copy(table.at[srcv.at[j]], buf, sem).wait()

        def scat(j, buf):
            pltpu.sync_copy(buf, acc.at[dstv.at[j]], add=True)
            if with_deg:
                pltpu.sync_copy(onesv, deg.at[dstv.at[j]], add=True)

        fire(0, buf0, sem0)

        def body(i, carry):
            j0 = 2 * i
            j1 = j0 + 1
            wait(j0, buf0, sem0)
            fire(j1, buf1, sem1)
            scat(j0, buf0)
            wait(j1, buf1, sem1)
            fire(j0 + 2, buf0, sem0)
            scat(j1, buf1)
            return carry
        lax.fori_loop(0, NPAIR, body, 0)
        wait(NCHUNK - 1, buf0, sem0)
        scat(NCHUNK - 1, buf0)

    for p in range(2):
        with_deg = p == 0
        pltpu.sync_copy(zer64, acc.at[stripe])
        if with_deg:
            pltpu.sync_copy(zer16, deg.at[stripe])
        plsc.subcore_barrier()

        @pl.when(c == 0)
        def _():
            edge_loop((x0, x1)[p], with_deg)

        @pl.when(c == 1)
        def _():
            edge_loop((x2, x3)[p], with_deg)

        plsc.subcore_barrier()

        zout = ((z0, z1), (z2, z3))

        @pl.when(c == 0)
        def _():
            pltpu.sync_copy(acc.at[stripe], zout[0][p].at[stripe])

        @pl.when(c == 1)
        def _():
            pltpu.sync_copy(acc.at[stripe], zout[1][p].at[stripe])

        plsc.subcore_barrier()

    @pl.when(c == 0)
    def _():
        pltpu.sync_copy(deg.at[stripe], degbuf)

        def rbody(r, carry):
            d = degbuf[r, :]
            degbuf[r, :] = 1.0 / jnp.maximum(d, 1.0)
            return carry
        lax.fori_loop(0, ROWS_PER_SUB, rbody, 0)
        pltpu.sync_copy(degbuf, recip16.at[stripe])


_agg1 = pl.kernel(
    _agg1_body,
    out_type=(
        jax.ShapeDtypeStruct((N_PAD, DSLAB), jnp.float32),
        jax.ShapeDtypeStruct((N_PAD, DSLAB), jnp.float32),
        jax.ShapeDtypeStruct((N_PAD, DSLAB), jnp.float32),
        jax.ShapeDtypeStruct((N_PAD, DSLAB), jnp.float32),
        jax.ShapeDtypeStruct((N_PAD, 16), jnp.float32),
    ),
    mesh=_MESH,
    compiler_params=pltpu.CompilerParams(use_tc_tiling_on_sc=False),
    scratch_types=[
        pltpu.VMEM_SHARED((N_PAD, DSLAB), jnp.float32),
        pltpu.VMEM_SHARED((N_PAD, 16), jnp.float32),
        pltpu.VMEM((NCHUNK, CHUNK), jnp.int32),
        pltpu.VMEM((NCHUNK, CHUNK), jnp.int32),
        pltpu.VMEM((CHUNK, DSLAB), jnp.float32),
        pltpu.VMEM((CHUNK, DSLAB), jnp.float32),
        pltpu.VMEM((CHUNK, 16), jnp.float32),
        pltpu.VMEM((ROWS_PER_SUB, 16), jnp.float32),
        pltpu.SemaphoreType.DMA,
        pltpu.SemaphoreType.DMA,
    ],
)


def _agg2_body(ylo, yhi, src3, dst3, recip16, b2, zer32,
               olo, ohi,
               acc, srcv, dstv, buf0, buf1, recipv, accbuf, b2v, sem0, sem1):
    c = lax.axis_index("c")
    s = lax.axis_index("s")
    row0 = s * ROWS_PER_SUB
    stripe = pl.ds(row0, ROWS_PER_SUB)

    pltpu.sync_copy(b2, b2v)
    pltpu.sync_copy(src3.at[s], srcv)
    pltpu.sync_copy(dst3.at[s], dstv)
    pltpu.sync_copy(zer32, acc.at[stripe])
    plsc.subcore_barrier()

    def edge_loop(table):
        def fire(j, buf, sem):
            pltpu.async_copy(table.at[srcv.at[j]], buf, sem)

        def wait(j, buf, sem):
            pltpu.make_async_copy(table.at[srcv.at[j]], buf, sem).wait()

        def scat(j, buf):
            pltpu.sync_copy(buf, acc.at[dstv.at[j]], add=True)

        fire(0, buf0, sem0)

        def body(i, carry):
            j0 = 2 * i
            j1 = j0 + 1
            wait(j0, buf0, sem0)
            fire(j1, buf1, sem1)
            scat(j0, buf0)
            wait(j1, buf1, sem1)
            fire(j0 + 2, buf0, sem0)
            scat(j1, buf1)
            return carry
        lax.fori_loop(0, NPAIR, body, 0)
        wait(NCHUNK - 1, buf0, sem0)
        scat(NCHUNK - 1, buf0)

    @pl.when(c == 0)
    def _():
        edge_loop(ylo)

    @pl.when(c == 1)
    def _():
        edge_loop(yhi)

    plsc.subcore_barrier()

    pltpu.sync_copy(acc.at[stripe], accbuf)
    pltpu.sync_copy(recip16.at[stripe], recipv)
    b2a = b2v[pl.ds(c * DY, 16)]
    b2b = b2v[pl.ds(c * DY + 16, 16)]

    def rbody(r, carry):
        rv = recipv[r, :]
        accbuf[r, pl.ds(0, 16)] = accbuf[r, pl.ds(0, 16)] * rv + b2a
        accbuf[r, pl.ds(16, 16)] = accbuf[r, pl.ds(16, 16)] * rv + b2b
        return carry
    lax.fori_loop(0, ROWS_PER_SUB, rbody, 0)

    @pl.when(c == 0)
    def _():
        pltpu.sync_copy(accbuf, olo.at[stripe])

    @pl.when(c == 1)
    def _():
        pltpu.sync_copy(accbuf, ohi.at[stripe])


_agg2 = pl.kernel(
    _agg2_body,
    out_type=(
        jax.ShapeDtypeStruct((N_PAD, DY), jnp.float32),
        jax.ShapeDtypeStruct((N_PAD, DY), jnp.float32),
    ),
    mesh=_MESH,
    compiler_params=pltpu.CompilerParams(use_tc_tiling_on_sc=False),
    scratch_types=[
        pltpu.VMEM_SHARED((N_PAD, DY), jnp.float32),
        pltpu.VMEM((NCHUNK, CHUNK), jnp.int32),
        pltpu.VMEM((NCHUNK, CHUNK), jnp.int32),
        pltpu.VMEM((CHUNK, DY), jnp.float32),
        pltpu.VMEM((CHUNK, DY), jnp.float32),
        pltpu.VMEM((ROWS_PER_SUB, 16), jnp.float32),
        pltpu.VMEM((ROWS_PER_SUB, DY), jnp.float32),
        pltpu.VMEM((2 * DY,), jnp.float32),
        pltpu.SemaphoreType.DMA,
        pltpu.SemaphoreType.DMA,
    ],
)


def _mlp_body(z0_ref, z1_ref, z2_ref, z3_ref, r16_ref, w1_ref, b1_ref,
              w2_ref, ylo_ref, yhi_ref):
    z = jnp.concatenate(
        [z0_ref[...], z1_ref[...], z2_ref[...], z3_ref[...]], axis=1)
    r = r16_ref[...][:, 0:1]
    h = jnp.dot(z * r, w1_ref[...], preferred_element_type=jnp.float32)
    h = jnp.maximum(h + b1_ref[...], 0.0)
    y = jnp.dot(h, w2_ref[...], preferred_element_type=jnp.float32)
    ylo_ref[...] = y[:, :DY]
    yhi_ref[...] = y[:, DY:]


def _mlp(z0, z1, z2, z3, recip16, W1, b1, W2):
    blk = 2048
    grid = (N_PAD // blk,)
    return pl.pallas_call(
        _mlp_body,
        grid=grid,
        in_specs=[
            pl.BlockSpec((blk, DSLAB), lambda i: (i, 0)),
            pl.BlockSpec((blk, DSLAB), lambda i: (i, 0)),
            pl.BlockSpec((blk, DSLAB), lambda i: (i, 0)),
            pl.BlockSpec((blk, DSLAB), lambda i: (i, 0)),
            pl.BlockSpec((blk, 16), lambda i: (i, 0)),
            pl.BlockSpec((256, 256), lambda i: (0, 0)),
            pl.BlockSpec((1, 256), lambda i: (0, 0)),
            pl.BlockSpec((256, 2 * DY), lambda i: (0, 0)),
        ],
        out_specs=[
            pl.BlockSpec((blk, DY), lambda i: (i, 0)),
            pl.BlockSpec((blk, DY), lambda i: (i, 0)),
        ],
        out_shape=[
            jax.ShapeDtypeStruct((N_PAD, DY), jnp.float32),
            jax.ShapeDtypeStruct((N_PAD, DY), jnp.float32),
        ],
    )(z0, z1, z2, z3, recip16, W1, b1, W2)


@jax.jit
def kernel(inputs, edge_index, W1, b1, W2, b2):
    src = edge_index[0].astype(jnp.int32)
    dst = edge_index[1].astype(jnp.int32)
    src3 = src.reshape(NSUB, NCHUNK, CHUNK)
    dst3 = dst.reshape(NSUB, NCHUNK, CHUNK)

    xs = [inputs[:, i * DSLAB:(i + 1) * DSLAB] for i in range(4)]
    zer64 = jnp.zeros((ROWS_PER_SUB, DSLAB), jnp.float32)
    zer16 = jnp.zeros((ROWS_PER_SUB, 16), jnp.float32)
    zer32 = jnp.zeros((ROWS_PER_SUB, DY), jnp.float32)
    ones16 = jnp.ones((CHUNK, 16), jnp.float32)

    z0, z1, z2, z3, recip16 = _agg1(
        xs[0], xs[1], xs[2], xs[3], src3, dst3, zer64, zer16, ones16)
    ylo, yhi = _mlp(z0, z1, z2, z3, recip16, W1, b1.reshape(1, -1), W2)
    olo, ohi = _agg2(ylo, yhi, src3, dst3, recip16, b2, zer32)
    return jnp.concatenate([olo, ohi], axis=1)[:N_NODES]


# speedup vs baseline: 8.1010x; 1.3896x over previous
import jax
import jax.numpy as jnp
from jax import lax
from jax.experimental import pallas as pl
from jax.experimental.pallas import tpu as pltpu
from jax.experimental.pallas import tpu_sc as plsc

N_NODES = 10000
N_PAD = 10240
N_EDGES = 160000
NCORE = 2
NSUB = 16
CHUNK = 200
EDGES_PER_SUB = N_EDGES // NSUB
NCHUNK = EDGES_PER_SUB // CHUNK
NBUF = 2
PRE = min(NBUF - 1, NCHUNK)
MAIN = ((NCHUNK - PRE) // NBUF) * NBUF
ROWS_PER_SUB = N_PAD // NSUB
DSLAB = 64
DY = 32

_MESH = plsc.VectorSubcoreMesh(core_axis_name="c", subcore_axis_name="s",
                               num_cores=NCORE, num_subcores=NSUB)


def _ring_loop(table, srcv, bufs, sems, scat):
    def fire(j, b):
        pltpu.async_copy(table.at[srcv.at[j]], bufs[b], sems[b])

    def wait(j, b):
        pltpu.make_async_copy(table.at[srcv.at[j]], bufs[b], sems[b]).wait()

    for j in range(PRE):
        fire(j, j % NBUF)

    def body(i, carry):
        for b in range(NBUF):
            j = NBUF * i + b
            wait(j, b)
            fire(j + PRE, (b + PRE) % NBUF)
            scat(j, bufs[b])
        return carry
    lax.fori_loop(0, MAIN // NBUF, body, 0)

    for j in range(MAIN, NCHUNK):
        wait(j, j % NBUF)
        if j + PRE < NCHUNK:
            fire(j + PRE, (j + PRE) % NBUF)
        scat(j, bufs[j % NBUF])


def _agg1_body(x0, x1, x2, x3, src3, dst3, zer64, zer16, ones16,
               z0, z1, z2, z3, recip16,
               acc, deg, srcv, dstv, buf0, buf1, onesv, degbuf, sem0, sem1):
    c = lax.axis_index("c")
    s = lax.axis_index("s")
    row0 = s * ROWS_PER_SUB
    stripe = pl.ds(row0, ROWS_PER_SUB)

    pltpu.sync_copy(ones16, onesv)
    pltpu.sync_copy(src3.at[s], srcv)
    pltpu.sync_copy(dst3.at[s], dstv)

    def edge_loop(table, with_deg):
        def scat(j, buf):
            pltpu.sync_copy(buf, acc.at[dstv.at[j]], add=True)
            if with_deg:
                pltpu.sync_copy(onesv, deg.at[dstv.at[j]], add=True)

        _ring_loop(table, srcv, (buf0, buf1), (sem0, sem1), scat)

    for p in range(2):
        with_deg = p == 0
        pltpu.sync_copy(zer64, acc.at[stripe])
        if with_deg:
            pltpu.sync_copy(zer16, deg.at[stripe])
        plsc.subcore_barrier()

        @pl.when(c == 0)
        def _():
            edge_loop((x0, x1)[p], with_deg)

        @pl.when(c == 1)
        def _():
            edge_loop((x2, x3)[p], with_deg)

        plsc.subcore_barrier()

        zout = ((z0, z1), (z2, z3))

        @pl.when(c == 0)
        def _():
            pltpu.sync_copy(acc.at[stripe], zout[0][p].at[stripe])

        @pl.when(c == 1)
        def _():
            pltpu.sync_copy(acc.at[stripe], zout[1][p].at[stripe])

        plsc.subcore_barrier()

    @pl.when(c == 0)
    def _():
        pltpu.sync_copy(deg.at[stripe], degbuf)

        def rbody(r, carry):
            d = degbuf[r, :]
            degbuf[r, :] = 1.0 / jnp.maximum(d, 1.0)
            return carry
        lax.fori_loop(0, ROWS_PER_SUB, rbody, 0)
        pltpu.sync_copy(degbuf, recip16.at[stripe])


_agg1 = pl.kernel(
    _agg1_body,
    out_type=(
        jax.ShapeDtypeStruct((N_PAD, DSLAB), jnp.float32),
        jax.ShapeDtypeStruct((N_PAD, DSLAB), jnp.float32),
        jax.ShapeDtypeStruct((N_PAD, DSLAB), jnp.float32),
        jax.ShapeDtypeStruct((N_PAD, DSLAB), jnp.float32),
        jax.ShapeDtypeStruct((N_PAD, 16), jnp.float32),
    ),
    mesh=_MESH,
    compiler_params=pltpu.CompilerParams(use_tc_tiling_on_sc=False),
    scratch_types=[
        pltpu.VMEM_SHARED((N_PAD, DSLAB), jnp.float32),
        pltpu.VMEM_SHARED((N_PAD, 16), jnp.float32),
        pltpu.VMEM((NCHUNK, CHUNK), jnp.int32),
        pltpu.VMEM((NCHUNK, CHUNK), jnp.int32),
        pltpu.VMEM((CHUNK, DSLAB), jnp.float32),
        pltpu.VMEM((CHUNK, DSLAB), jnp.float32),
        pltpu.VMEM((CHUNK, 16), jnp.float32),
        pltpu.VMEM((ROWS_PER_SUB, 16), jnp.float32),
        pltpu.SemaphoreType.DMA,
        pltpu.SemaphoreType.DMA,
    ],
)


def _agg2_body(ylo, yhi, src3, dst3, recip16, b2, zer32,
               olo, ohi,
               acc, srcv, dstv, buf0, buf1, recipv, accbuf, b2v, sem0, sem1):
    c = lax.axis_index("c")
    s = lax.axis_index("s")
    row0 = s * ROWS_PER_SUB
    stripe = pl.ds(row0, ROWS_PER_SUB)

    pltpu.sync_copy(b2, b2v)
    pltpu.sync_copy(src3.at[s], srcv)
    pltpu.sync_copy(dst3.at[s], dstv)
    pltpu.sync_copy(zer32, acc.at[stripe])
    plsc.subcore_barrier()

    def edge_loop(table):
        def scat(j, buf):
            pltpu.sync_copy(buf, acc.at[dstv.at[j]], add=True)

        _ring_loop(table, srcv, (buf0, buf1), (sem0, sem1), scat)

    @pl.when(c == 0)
    def _():
        edge_loop(ylo)

    @pl.when(c == 1)
    def _():
        edge_loop(yhi)

    plsc.subcore_barrier()

    pltpu.sync_copy(acc.at[stripe], accbuf)
    pltpu.sync_copy(recip16.at[stripe], recipv)
    b2a = b2v[pl.ds(c * DY, 16)]
    b2b = b2v[pl.ds(c * DY + 16, 16)]

    def rbody(r, carry):
        rv = recipv[r, :]
        accbuf[r, pl.ds(0, 16)] = accbuf[r, pl.ds(0, 16)] * rv + b2a
        accbuf[r, pl.ds(16, 16)] = accbuf[r, pl.ds(16, 16)] * rv + b2b
        return carry
    lax.fori_loop(0, ROWS_PER_SUB, rbody, 0)

    @pl.when(c == 0)
    def _():
        pltpu.sync_copy(accbuf, olo.at[stripe])

    @pl.when(c == 1)
    def _():
        pltpu.sync_copy(accbuf, ohi.at[stripe])


_agg2 = pl.kernel(
    _agg2_body,
    out_type=(
        jax.ShapeDtypeStruct((N_PAD, DY), jnp.float32),
        jax.ShapeDtypeStruct((N_PAD, DY), jnp.float32),
    ),
    mesh=_MESH,
    compiler_params=pltpu.CompilerParams(use_tc_tiling_on_sc=False),
    scratch_types=[
        pltpu.VMEM_SHARED((N_PAD, DY), jnp.float32),
        pltpu.VMEM((NCHUNK, CHUNK), jnp.int32),
        pltpu.VMEM((NCHUNK, CHUNK), jnp.int32),
        pltpu.VMEM((CHUNK, DY), jnp.float32),
        pltpu.VMEM((CHUNK, DY), jnp.float32),
        pltpu.VMEM((ROWS_PER_SUB, 16), jnp.float32),
        pltpu.VMEM((ROWS_PER_SUB, DY), jnp.float32),
        pltpu.VMEM((2 * DY,), jnp.float32),
        pltpu.SemaphoreType.DMA,
        pltpu.SemaphoreType.DMA,
    ],
)


def _mlp_body(z0_ref, z1_ref, z2_ref, z3_ref, r16_ref, w1_ref, b1_ref,
              w2_ref, ylo_ref, yhi_ref):
    z = jnp.concatenate(
        [z0_ref[...], z1_ref[...], z2_ref[...], z3_ref[...]], axis=1)
    r = r16_ref[...][:, 0:1]
    h = jnp.dot(z * r, w1_ref[...], preferred_element_type=jnp.float32)
    h = jnp.maximum(h + b1_ref[...], 0.0)
    y = jnp.dot(h, w2_ref[...], preferred_element_type=jnp.float32)
    ylo_ref[...] = y[:, :DY]
    yhi_ref[...] = y[:, DY:]


def _mlp(z0, z1, z2, z3, recip16, W1, b1, W2):
    blk = 2048
    grid = (N_PAD // blk,)
    return pl.pallas_call(
        _mlp_body,
        grid=grid,
        in_specs=[
            pl.BlockSpec((blk, DSLAB), lambda i: (i, 0)),
            pl.BlockSpec((blk, DSLAB), lambda i: (i, 0)),
            pl.BlockSpec((blk, DSLAB), lambda i: (i, 0)),
            pl.BlockSpec((blk, DSLAB), lambda i: (i, 0)),
            pl.BlockSpec((blk, 16), lambda i: (i, 0)),
            pl.BlockSpec((256, 256), lambda i: (0, 0)),
            pl.BlockSpec((1, 256), lambda i: (0, 0)),
            pl.BlockSpec((256, 2 * DY), lambda i: (0, 0)),
        ],
        out_specs=[
            pl.BlockSpec((blk, DY), lambda i: (i, 0)),
            pl.BlockSpec((blk, DY), lambda i: (i, 0)),
        ],
        out_shape=[
            jax.ShapeDtypeStruct((N_PAD, DY), jnp.float32),
            jax.ShapeDtypeStruct((N_PAD, DY), jnp.float32),
        ],
    )(z0, z1, z2, z3, recip16, W1, b1, W2)


@jax.jit
def kernel(inputs, edge_index, W1, b1, W2, b2):
    src = edge_index[0].astype(jnp.int32)
    dst = edge_index[1].astype(jnp.int32)
    src3 = src.reshape(NSUB, NCHUNK, CHUNK)
    dst3 = dst.reshape(NSUB, NCHUNK, CHUNK)

    xs = [inputs[:, i * DSLAB:(i + 1) * DSLAB] for i in range(4)]
    zer64 = jnp.zeros((ROWS_PER_SUB, DSLAB), jnp.float32)
    zer16 = jnp.zeros((ROWS_PER_SUB, 16), jnp.float32)
    zer32 = jnp.zeros((ROWS_PER_SUB, DY), jnp.float32)
    ones16 = jnp.ones((CHUNK, 16), jnp.float32)

    z0, z1, z2, z3, recip16 = _agg1(
        xs[0], xs[1], xs[2], xs[3], src3, dst3, zer64, zer16, ones16)
    ylo, yhi = _mlp(z0, z1, z2, z3, recip16, W1, b1.reshape(1, -1), W2)
    olo, ohi = _agg2(ylo, yhi, src3, dst3, recip16, b2, zer32)
    return jnp.concatenate([olo, ohi], axis=1)[:N_NODES]


# speedup vs baseline: 9.7000x; 1.1974x over previous
import jax
import jax.numpy as jnp
from jax import lax
from jax.experimental import pallas as pl
from jax.experimental.pallas import tpu as pltpu
from jax.experimental.pallas import tpu_sc as plsc

N_NODES = 10000
N_PAD = 10240
N_EDGES = 160000
NCORE = 2
NSUB = 16
CHUNK = 200
EDGES_PER_SUB = N_EDGES // NSUB
NCHUNK = EDGES_PER_SUB // CHUNK
NBUF = 3
PRE = min(NBUF - 1, NCHUNK)
MAIN = ((NCHUNK - PRE) // NBUF) * NBUF
ROWS_PER_SUB = N_PAD // NSUB
DSLAB = 64
DY = 32

_MESH = plsc.VectorSubcoreMesh(core_axis_name="c", subcore_axis_name="s",
                               num_cores=NCORE, num_subcores=NSUB)


def _ring_loop(table, srcv, bufs, sems, scat):
    def fire(j, b):
        pltpu.async_copy(table.at[srcv.at[j]], bufs[b], sems[b])

    def wait(j, b):
        pltpu.make_async_copy(table.at[srcv.at[j]], bufs[b], sems[b]).wait()

    for j in range(PRE):
        fire(j, j % NBUF)

    def body(i, carry):
        for b in range(NBUF):
            j = NBUF * i + b
            wait(j, b)
            fire(j + PRE, (b + PRE) % NBUF)
            scat(j, bufs[b])
        return carry
    lax.fori_loop(0, MAIN // NBUF, body, 0)

    for j in range(MAIN, NCHUNK):
        wait(j, j % NBUF)
        if j + PRE < NCHUNK:
            fire(j + PRE, (j + PRE) % NBUF)
        scat(j, bufs[j % NBUF])


def _agg1_body(x0, x1, x2, x3, src3, dst3, zer64, zer16, ones16,
               z0, z1, z2, z3, recip16,
               acc, deg, srcv, dstv, buf0, buf1, buf2, onesv, degbuf,
               sem0, sem1, sem2):
    c = lax.axis_index("c")
    s = lax.axis_index("s")
    row0 = s * ROWS_PER_SUB
    stripe = pl.ds(row0, ROWS_PER_SUB)

    pltpu.sync_copy(ones16, onesv)
    pltpu.sync_copy(src3.at[s], srcv)
    pltpu.sync_copy(dst3.at[s], dstv)

    def edge_loop(table, with_deg):
        def scat(j, buf):
            pltpu.sync_copy(buf, acc.at[dstv.at[j]], add=True)
            if with_deg:
                pltpu.sync_copy(onesv, deg.at[dstv.at[j]], add=True)

        _ring_loop(table, srcv, (buf0, buf1, buf2), (sem0, sem1, sem2), scat)

    for p in range(2):
        with_deg = p == 0
        pltpu.sync_copy(zer64, acc.at[stripe])
        if with_deg:
            pltpu.sync_copy(zer16, deg.at[stripe])
        plsc.subcore_barrier()

        @pl.when(c == 0)
        def _():
            edge_loop((x0, x1)[p], with_deg)

        @pl.when(c == 1)
        def _():
            edge_loop((x2, x3)[p], with_deg)

        plsc.subcore_barrier()

        zout = ((z0, z1), (z2, z3))

        @pl.when(c == 0)
        def _():
            pltpu.sync_copy(acc.at[stripe], zout[0][p].at[stripe])

        @pl.when(c == 1)
        def _():
            pltpu.sync_copy(acc.at[stripe], zout[1][p].at[stripe])

        plsc.subcore_barrier()

    @pl.when(c == 0)
    def _():
        pltpu.sync_copy(deg.at[stripe], degbuf)

        def rbody(r, carry):
            d = degbuf[r, :]
            degbuf[r, :] = 1.0 / jnp.maximum(d, 1.0)
            return carry
        lax.fori_loop(0, ROWS_PER_SUB, rbody, 0)
        pltpu.sync_copy(degbuf, recip16.at[stripe])


_agg1 = pl.kernel(
    _agg1_body,
    out_type=(
        jax.ShapeDtypeStruct((N_PAD, DSLAB), jnp.float32),
        jax.ShapeDtypeStruct((N_PAD, DSLAB), jnp.float32),
        jax.ShapeDtypeStruct((N_PAD, DSLAB), jnp.float32),
        jax.ShapeDtypeStruct((N_PAD, DSLAB), jnp.float32),
        jax.ShapeDtypeStruct((N_PAD, 16), jnp.float32),
    ),
    mesh=_MESH,
    compiler_params=pltpu.CompilerParams(use_tc_tiling_on_sc=False),
    scratch_types=[
        pltpu.VMEM_SHARED((N_PAD, DSLAB), jnp.float32),
        pltpu.VMEM_SHARED((N_PAD, 16), jnp.float32),
        pltpu.VMEM((NCHUNK, CHUNK), jnp.int32),
        pltpu.VMEM((NCHUNK, CHUNK), jnp.int32),
        pltpu.VMEM((CHUNK, DSLAB), jnp.float32),
        pltpu.VMEM((CHUNK, DSLAB), jnp.float32),
        pltpu.VMEM((CHUNK, DSLAB), jnp.float32),
        pltpu.VMEM((CHUNK, 16), jnp.float32),
        pltpu.VMEM((ROWS_PER_SUB, 16), jnp.float32),
        pltpu.SemaphoreType.DMA,
        pltpu.SemaphoreType.DMA,
        pltpu.SemaphoreType.DMA,
    ],
)


def _agg2_body(ylo, yhi, src3, dst3, recip16, b2, zer32,
               olo, ohi,
               acc, srcv, dstv, buf0, buf1, buf2, recipv, accbuf, b2v,
               sem0, sem1, sem2):
    c = lax.axis_index("c")
    s = lax.axis_index("s")
    row0 = s * ROWS_PER_SUB
    stripe = pl.ds(row0, ROWS_PER_SUB)

    pltpu.sync_copy(b2, b2v)
    pltpu.sync_copy(src3.at[s], srcv)
    pltpu.sync_copy(dst3.at[s], dstv)
    pltpu.sync_copy(zer32, acc.at[stripe])
    plsc.subcore_barrier()

    def edge_loop(table):
        def scat(j, buf):
            pltpu.sync_copy(buf, acc.at[dstv.at[j]], add=True)

        _ring_loop(table, srcv, (buf0, buf1, buf2), (sem0, sem1, sem2), scat)

    @pl.when(c == 0)
    def _():
        edge_loop(ylo)

    @pl.when(c == 1)
    def _():
        edge_loop(yhi)

    plsc.subcore_barrier()

    pltpu.sync_copy(acc.at[stripe], accbuf)
    pltpu.sync_copy(recip16.at[stripe], recipv)
    b2a = b2v[pl.ds(c * DY, 16)]
    b2b = b2v[pl.ds(c * DY + 16, 16)]

    def rbody(r, carry):
        rv = recipv[r, :]
        accbuf[r, pl.ds(0, 16)] = accbuf[r, pl.ds(0, 16)] * rv + b2a
        accbuf[r, pl.ds(16, 16)] = accbuf[r, pl.ds(16, 16)] * rv + b2b
        return carry
    lax.fori_loop(0, ROWS_PER_SUB, rbody, 0)

    @pl.when(c == 0)
    def _():
        pltpu.sync_copy(accbuf, olo.at[stripe])

    @pl.when(c == 1)
    def _():
        pltpu.sync_copy(accbuf, ohi.at[stripe])


_agg2 = pl.kernel(
    _agg2_body,
    out_type=(
        jax.ShapeDtypeStruct((N_PAD, DY), jnp.float32),
        jax.ShapeDtypeStruct((N_PAD, DY), jnp.float32),
    ),
    mesh=_MESH,
    compiler_params=pltpu.CompilerParams(use_tc_tiling_on_sc=False),
    scratch_types=[
        pltpu.VMEM_SHARED((N_PAD, DY), jnp.float32),
        pltpu.VMEM((NCHUNK, CHUNK), jnp.int32),
        pltpu.VMEM((NCHUNK, CHUNK), jnp.int32),
        pltpu.VMEM((CHUNK, DY), jnp.float32),
        pltpu.VMEM((CHUNK, DY), jnp.float32),
        pltpu.VMEM((CHUNK, DY), jnp.float32),
        pltpu.VMEM((ROWS_PER_SUB, 16), jnp.float32),
        pltpu.VMEM((ROWS_PER_SUB, DY), jnp.float32),
        pltpu.VMEM((2 * DY,), jnp.float32),
        pltpu.SemaphoreType.DMA,
        pltpu.SemaphoreType.DMA,
        pltpu.SemaphoreType.DMA,
    ],
)


def _mlp_body(z0_ref, z1_ref, z2_ref, z3_ref, r16_ref, w1_ref, b1_ref,
              w2_ref, ylo_ref, yhi_ref):
    z = jnp.concatenate(
        [z0_ref[...], z1_ref[...], z2_ref[...], z3_ref[...]], axis=1)
    r = r16_ref[...][:, 0:1]
    h = jnp.dot(z * r, w1_ref[...], preferred_element_type=jnp.float32)
    h = jnp.maximum(h + b1_ref[...], 0.0)
    y = jnp.dot(h, w2_ref[...], preferred_element_type=jnp.float32)
    ylo_ref[...] = y[:, :DY]
    yhi_ref[...] = y[:, DY:]


def _mlp(z0, z1, z2, z3, recip16, W1, b1, W2):
    blk = 2048
    grid = (N_PAD // blk,)
    return pl.pallas_call(
        _mlp_body,
        grid=grid,
        in_specs=[
            pl.BlockSpec((blk, DSLAB), lambda i: (i, 0)),
            pl.BlockSpec((blk, DSLAB), lambda i: (i, 0)),
            pl.BlockSpec((blk, DSLAB), lambda i: (i, 0)),
            pl.BlockSpec((blk, DSLAB), lambda i: (i, 0)),
            pl.BlockSpec((blk, 16), lambda i: (i, 0)),
            pl.BlockSpec((256, 256), lambda i: (0, 0)),
            pl.BlockSpec((1, 256), lambda i: (0, 0)),
            pl.BlockSpec((256, 2 * DY), lambda i: (0, 0)),
        ],
        out_specs=[
            pl.BlockSpec((blk, DY), lambda i: (i, 0)),
            pl.BlockSpec((blk, DY), lambda i: (i, 0)),
        ],
        out_shape=[
            jax.ShapeDtypeStruct((N_PAD, DY), jnp.float32),
            jax.ShapeDtypeStruct((N_PAD, DY), jnp.float32),
        ],
    )(z0, z1, z2, z3, recip16, W1, b1, W2)


@jax.jit
def kernel(inputs, edge_index, W1, b1, W2, b2):
    src = edge_index[0].astype(jnp.int32)
    dst = edge_index[1].astype(jnp.int32)
    src3 = src.reshape(NSUB, NCHUNK, CHUNK)
    dst3 = dst.reshape(NSUB, NCHUNK, CHUNK)

    xs = [inputs[:, i * DSLAB:(i + 1) * DSLAB] for i in range(4)]
    zer64 = jnp.zeros((ROWS_PER_SUB, DSLAB), jnp.float32)
    zer16 = jnp.zeros((ROWS_PER_SUB, 16), jnp.float32)
    zer32 = jnp.zeros((ROWS_PER_SUB, DY), jnp.float32)
    ones16 = jnp.ones((CHUNK, 16), jnp.float32)

    z0, z1, z2, z3, recip16 = _agg1(
        xs[0], xs[1], xs[2], xs[3], src3, dst3, zer64, zer16, ones16)
    ylo, yhi = _mlp(z0, z1, z2, z3, recip16, W1, b1.reshape(1, -1), W2)
    olo, ohi = _agg2(ylo, yhi, src3, dst3, recip16, b2, zer32)
    return jnp.concatenate([olo, ohi], axis=1)[:N_NODES]


# speedup vs baseline: 11.5054x; 1.1861x over previous
import jax
import jax.numpy as jnp
from jax import lax
from jax.experimental import pallas as pl
from jax.experimental.pallas import tpu as pltpu
from jax.experimental.pallas import tpu_sc as plsc

N_NODES = 10000
N_PAD = 10240
N_EDGES = 160000
NCORE = 2
NSUB = 16
CHUNK = 200
EDGES_PER_SUB = N_EDGES // NSUB
NCHUNK = EDGES_PER_SUB // CHUNK
NBUF = 3
PRE = min(NBUF - 1, NCHUNK)
MAIN = ((NCHUNK - PRE) // NBUF) * NBUF
ROWS_PER_SUB = N_PAD // NSUB
DIN = 256
DSLAB = 64
DY = 32

_MESH = plsc.VectorSubcoreMesh(core_axis_name="c", subcore_axis_name="s",
                               num_cores=NCORE, num_subcores=NSUB)


def _ring_loop(gather_slice, srcv, bufs, sems, scat):
    def fire(j, b):
        pltpu.async_copy(gather_slice(j), bufs[b], sems[b])

    def wait(j, b):
        pltpu.make_async_copy(gather_slice(j), bufs[b], sems[b]).wait()

    for j in range(PRE):
        fire(j, j % NBUF)

    def body(i, carry):
        for b in range(NBUF):
            j = NBUF * i + b
            wait(j, b)
            fire(j + PRE, (b + PRE) % NBUF)
            scat(j, bufs[b])
        return carry
    lax.fori_loop(0, MAIN // NBUF, body, 0)

    for j in range(MAIN, NCHUNK):
        wait(j, j % NBUF)
        if j + PRE < NCHUNK:
            fire(j + PRE, (j + PRE) % NBUF)
        scat(j, bufs[j % NBUF])


def _agg1_body(x4, src4q, dst3, zer64, zer16, ones16,
               z, recip16,
               acc, deg, srcv, dstv, buf0, buf1, buf2, onesv, degbuf,
               sem0, sem1, sem2):
    c = lax.axis_index("c")
    s = lax.axis_index("s")
    row0 = s * ROWS_PER_SUB
    stripe = pl.ds(row0, ROWS_PER_SUB)

    pltpu.sync_copy(ones16, onesv)
    pltpu.sync_copy(dst3.at[s], dstv)

    def edge_loop(with_deg):
        def gather_slice(j):
            return x4.at[srcv.at[j]]

        def scat(j, buf):
            pltpu.sync_copy(buf, acc.at[dstv.at[j]], add=True)
            if with_deg:
                pltpu.sync_copy(onesv, deg.at[dstv.at[j]], add=True)

        _ring_loop(gather_slice, srcv, (buf0, buf1, buf2),
                   (sem0, sem1, sem2), scat)

    for p in range(2):
        with_deg = p == 0
        pltpu.sync_copy(zer64, acc.at[stripe])
        if with_deg:
            pltpu.sync_copy(zer16, deg.at[stripe])

        @pl.when(c == 0)
        def _():
            pltpu.sync_copy(src4q.at[p, s], srcv)

        @pl.when(c == 1)
        def _():
            pltpu.sync_copy(src4q.at[2 + p, s], srcv)

        plsc.subcore_barrier()

        edge_loop(with_deg)

        plsc.subcore_barrier()

        @pl.when(c == 0)
        def _():
            pltpu.sync_copy(acc.at[stripe],
                            z.at[stripe, pl.ds(p * DSLAB, DSLAB)])

        @pl.when(c == 1)
        def _():
            pltpu.sync_copy(acc.at[stripe],
                            z.at[stripe, pl.ds(2 * DSLAB + p * DSLAB, DSLAB)])

        plsc.subcore_barrier()

    @pl.when(c == 0)
    def _():
        pltpu.sync_copy(deg.at[stripe], degbuf)

        def rbody(r, carry):
            d = degbuf[r, :]
            degbuf[r, :] = 1.0 / jnp.maximum(d, 1.0)
            return carry
        lax.fori_loop(0, ROWS_PER_SUB, rbody, 0)
        pltpu.sync_copy(degbuf, recip16.at[stripe])


_agg1 = pl.kernel(
    _agg1_body,
    out_type=(
        jax.ShapeDtypeStruct((N_PAD, DIN), jnp.float32),
        jax.ShapeDtypeStruct((N_PAD, 16), jnp.float32),
    ),
    mesh=_MESH,
    compiler_params=pltpu.CompilerParams(use_tc_tiling_on_sc=False),
    scratch_types=[
        pltpu.VMEM_SHARED((N_PAD, DSLAB), jnp.float32),
        pltpu.VMEM_SHARED((N_PAD, 16), jnp.float32),
        pltpu.VMEM((NCHUNK, CHUNK), jnp.int32),
        pltpu.VMEM((NCHUNK, CHUNK), jnp.int32),
        pltpu.VMEM((CHUNK, DSLAB), jnp.float32),
        pltpu.VMEM((CHUNK, DSLAB), jnp.float32),
        pltpu.VMEM((CHUNK, DSLAB), jnp.float32),
        pltpu.VMEM((CHUNK, 16), jnp.float32),
        pltpu.VMEM((ROWS_PER_SUB, 16), jnp.float32),
        pltpu.SemaphoreType.DMA,
        pltpu.SemaphoreType.DMA,
        pltpu.SemaphoreType.DMA,
    ],
)


def _agg2_body(y2, src2q, dst3, recip16, b2, zer32,
               out,
               acc, srcv, dstv, buf0, buf1, buf2, recipv, accbuf, b2v,
               sem0, sem1, sem2):
    c = lax.axis_index("c")
    s = lax.axis_index("s")
    row0 = s * ROWS_PER_SUB
    stripe = pl.ds(row0, ROWS_PER_SUB)

    pltpu.sync_copy(b2, b2v)
    pltpu.sync_copy(dst3.at[s], dstv)
    pltpu.sync_copy(zer32, acc.at[stripe])

    @pl.when(c == 0)
    def _():
        pltpu.sync_copy(src2q.at[0, s], srcv)

    @pl.when(c == 1)
    def _():
        pltpu.sync_copy(src2q.at[1, s], srcv)

    plsc.subcore_barrier()

    def edge_loop():
        def gather_slice(j):
            return y2.at[srcv.at[j]]

        def scat(j, buf):
            pltpu.sync_copy(buf, acc.at[dstv.at[j]], add=True)

        _ring_loop(gather_slice, srcv, (buf0, buf1, buf2),
                   (sem0, sem1, sem2), scat)

    edge_loop()

    plsc.subcore_barrier()

    pltpu.sync_copy(acc.at[stripe], accbuf)
    pltpu.sync_copy(recip16.at[stripe], recipv)
    b2a = b2v[pl.ds(c * DY, 16)]
    b2b = b2v[pl.ds(c * DY + 16, 16)]

    def rbody(r, carry):
        rv = recipv[r, :]
        accbuf[r, pl.ds(0, 16)] = accbuf[r, pl.ds(0, 16)] * rv + b2a
        accbuf[r, pl.ds(16, 16)] = accbuf[r, pl.ds(16, 16)] * rv + b2b
        return carry
    lax.fori_loop(0, ROWS_PER_SUB, rbody, 0)

    @pl.when(c == 0)
    def _():
        pltpu.sync_copy(accbuf, out.at[stripe, pl.ds(0, DY)])

    @pl.when(c == 1)
    def _():
        pltpu.sync_copy(accbuf, out.at[stripe, pl.ds(DY, DY)])


_agg2 = pl.kernel(
    _agg2_body,
    out_type=(
        jax.ShapeDtypeStruct((N_PAD, 2 * DY), jnp.float32),
    ),
    mesh=_MESH,
    compiler_params=pltpu.CompilerParams(use_tc_tiling_on_sc=False),
    scratch_types=[
        pltpu.VMEM_SHARED((N_PAD, DY), jnp.float32),
        pltpu.VMEM((NCHUNK, CHUNK), jnp.int32),
        pltpu.VMEM((NCHUNK, CHUNK), jnp.int32),
        pltpu.VMEM((CHUNK, DY), jnp.float32),
        pltpu.VMEM((CHUNK, DY), jnp.float32),
        pltpu.VMEM((CHUNK, DY), jnp.float32),
        pltpu.VMEM((ROWS_PER_SUB, 16), jnp.float32),
        pltpu.VMEM((ROWS_PER_SUB, DY), jnp.float32),
        pltpu.VMEM((2 * DY,), jnp.float32),
        pltpu.SemaphoreType.DMA,
        pltpu.SemaphoreType.DMA,
        pltpu.SemaphoreType.DMA,
    ],
)


def _mlp_body(z_ref, r16_ref, w1_ref, b1_ref, w2_ref, y_ref):
    z = z_ref[...]
    r = r16_ref[...][:, 0:1]
    h = jnp.dot(z * r, w1_ref[...], preferred_element_type=jnp.float32)
    h = jnp.maximum(h + b1_ref[...], 0.0)
    y_ref[...] = jnp.dot(h, w2_ref[...], preferred_element_type=jnp.float32)


def _mlp(z, recip16, W1, b1, W2):
    blk = 2048
    grid = (N_PAD // blk,)
    return pl.pallas_call(
        _mlp_body,
        grid=grid,
        in_specs=[
            pl.BlockSpec((blk, DIN), lambda i: (i, 0)),
            pl.BlockSpec((blk, 16), lambda i: (i, 0)),
            pl.BlockSpec((DIN, DIN), lambda i: (0, 0)),
            pl.BlockSpec((1, DIN), lambda i: (0, 0)),
            pl.BlockSpec((DIN, 2 * DY), lambda i: (0, 0)),
        ],
        out_specs=pl.BlockSpec((blk, 2 * DY), lambda i: (i, 0)),
        out_shape=jax.ShapeDtypeStruct((N_PAD, 2 * DY), jnp.float32),
    )(z, recip16, W1, b1, W2)


@jax.jit
def kernel(inputs, edge_index, W1, b1, W2, b2):
    src = edge_index[0].astype(jnp.int32)
    dst = edge_index[1].astype(jnp.int32)
    dst3 = dst.reshape(NSUB, NCHUNK, CHUNK)

    src4 = (src * 4).reshape(NSUB, NCHUNK, CHUNK)
    src4q = jnp.stack([src4, src4 + 1, src4 + 2, src4 + 3])
    src2 = (src * 2).reshape(NSUB, NCHUNK, CHUNK)
    src2q = jnp.stack([src2, src2 + 1])

    x4 = inputs.reshape(4 * N_NODES, DSLAB)

    zer64 = jnp.zeros((ROWS_PER_SUB, DSLAB), jnp.float32)
    zer16 = jnp.zeros((ROWS_PER_SUB, 16), jnp.float32)
    zer32 = jnp.zeros((ROWS_PER_SUB, DY), jnp.float32)
    ones16 = jnp.ones((CHUNK, 16), jnp.float32)

    z, recip16 = _agg1(x4, src4q, dst3, zer64, zer16, ones16)
    y = _mlp(z, recip16, W1, b1.reshape(1, -1), W2)
    y2 = y.reshape(2 * N_PAD, DY)
    out = _agg2(y2, src2q, dst3, recip16, b2, zer32)
    return out[:N_NODES]
